# baseline (speedup 1.0000x reference)
import functools

import jax
import jax.numpy as jnp
from jax import lax
from jax.experimental import pallas as pl
from jax.experimental.pallas import tpu as pltpu
from jax.experimental.pallas import tpu_sc as plsc

LANES = 16
NC = 2
NS = 16
NW = NC * NS
NBUF = 4
DG = 2


@functools.lru_cache(maxsize=None)
def _make_emb(batch, maxlen, embed):
    RPW = batch // NW
    assert batch % NW == 0 and RPW % NBUF == 0 and embed == 2 * LANES

    mesh = plsc.VectorSubcoreMesh(core_axis_name="c", subcore_axis_name="s")

    @functools.partial(
        pl.kernel,
        mesh=mesh,
        compiler_params=pltpu.CompilerParams(use_tc_tiling_on_sc=False),
        out_type=jax.ShapeDtypeStruct((batch, maxlen, embed), jnp.float32),
        scratch_types=(
            [pltpu.VMEM((RPW, maxlen), jnp.int32),
             pltpu.VMEM((maxlen, embed), jnp.float32)]
            + [pltpu.VMEM((maxlen, embed), jnp.float32) for _ in range(NBUF)]
            + [pltpu.SemaphoreType.DMA for _ in range(2 * NBUF + 1)]
        ),
    )
    def emb(x_hbm, tok_hbm, pos_hbm, out_hbm, idx_v, pat_v, *rest):
        rows = rest[:NBUF]
        gsems = rest[NBUF:2 * NBUF]
        ssems = rest[2 * NBUF:3 * NBUF]
        lsem = rest[3 * NBUF]

        wid = lax.axis_index("s") * NC + lax.axis_index("c")
        base = wid * RPW

        pltpu.async_copy(x_hbm.at[pl.ds(base, RPW)], idx_v, lsem).wait()
        pltpu.async_copy(pos_hbm, pat_v, lsem).wait()

        def start_gather(k, b):
            pltpu.make_async_copy(
                tok_hbm.at[idx_v.at[k]], rows[b], gsems[b]
            ).start()

        def wait_gather(b):
            pltpu.make_async_copy(
                tok_hbm.at[idx_v.at[0]], rows[b], gsems[b]
            ).wait()

        def start_scatter(k, b):
            pltpu.make_async_copy(rows[b], out_hbm.at[base + k], ssems[b]).start()

        def wait_scatter(b):
            pltpu.make_async_copy(rows[b], out_hbm.at[base], ssems[b]).wait()

        def add_pattern(rows_ref):
            def body(r, carry):
                plsc.addupdate(rows_ref.at[r, pl.ds(0, LANES)],
                               pat_v[r, pl.ds(0, LANES)])
                plsc.addupdate(rows_ref.at[r, pl.ds(LANES, LANES)],
                               pat_v[r, pl.ds(LANES, LANES)])
                return carry
            lax.fori_loop(0, maxlen, body, 0)

        for b in range(DG):
            start_gather(b, b)

        def outer(i, carry):
            k0 = i * NBUF
            for b in range(NBUF):
                k = k0 + b
                nxt = k + DG
                bn = (b + DG) % NBUF

                @pl.when(nxt < RPW)
                def _(nxt=nxt, bn=bn):
                    @pl.when(nxt >= NBUF)
                    def _():
                        wait_scatter(bn)
                    start_gather(nxt, bn)

                wait_gather(b)
                add_pattern(rows[b])
                start_scatter(k, b)
            return carry

        lax.fori_loop(0, RPW // NBUF, outer, 0)

        for b in range(NBUF):
            wait_scatter(b)

    return emb


def kernel(x, token_table, pos_table):
    batch, maxlen = x.shape
    embed = token_table.shape[-1]
    return _make_emb(batch, maxlen, embed)(
        x.astype(jnp.int32), token_table, pos_table
    )
